# Initial kernel scaffold; baseline (speedup 1.0000x reference)
import jax
import jax.numpy as jnp
from jax.experimental import pallas as pl


def kernel(indices, codebook):
    raise NotImplementedError("write your pallas kernel here")


# speedup vs baseline: 2.9840x; 2.9840x over previous
import functools

import jax
import jax.numpy as jnp
from jax import lax
from jax.experimental import pallas as pl
from jax.experimental.pallas import tpu as pltpu
from jax.experimental.pallas import tpu_sc as plsc

_NUM_CORES = 2
_NUM_SUBCORES = 16
_NW = _NUM_CORES * _NUM_SUBCORES
_CHUNK = 128


@functools.partial(jax.jit, static_argnames=())
def _sc_gather(idx_flat, codebook):
    B = idx_flat.shape[0]
    V, D = codebook.shape
    b_per_w = B // _NW
    n_chunks = b_per_w // _CHUNK
    mesh = plsc.VectorSubcoreMesh(core_axis_name="c", subcore_axis_name="s")

    @functools.partial(
        pl.kernel,
        out_type=jax.ShapeDtypeStruct((B, D), jnp.float32),
        mesh=mesh,
        scratch_types=[
            pltpu.VMEM((b_per_w,), jnp.int32),
            pltpu.VMEM((_CHUNK, D), jnp.float32),
            pltpu.SemaphoreType.DMA,
        ],
    )
    def k(table_hbm, idx_hbm, out_hbm, idx_v, rows_v, sem):
        wid = lax.axis_index("s") * _NUM_CORES + lax.axis_index("c")
        base = wid * b_per_w
        pltpu.sync_copy(idx_hbm.at[pl.ds(base, b_per_w)], idx_v)

        @pl.loop(0, n_chunks)
        def _(g):
            off = g * _CHUNK
            pltpu.async_copy(
                table_hbm.at[idx_v.at[pl.ds(off, _CHUNK)]], rows_v, sem
            ).wait()
            pltpu.sync_copy(rows_v, out_hbm.at[pl.ds(base + off, _CHUNK)])

    return k(codebook, idx_flat)


def kernel(indices, codebook):
    shape = indices.shape
    idx_flat = indices.reshape(-1).astype(jnp.int32)
    out = _sc_gather(idx_flat, codebook.astype(jnp.float32))
    return out.reshape(*shape, codebook.shape[1])


# speedup vs baseline: 3.0638x; 1.0267x over previous
import functools

import jax
import jax.numpy as jnp
from jax import lax
from jax.experimental import pallas as pl
from jax.experimental.pallas import tpu as pltpu
from jax.experimental.pallas import tpu_sc as plsc

_NUM_CORES = 2
_NUM_SUBCORES = 16
_NW = _NUM_CORES * _NUM_SUBCORES
_CHUNK = 64


@functools.partial(jax.jit, static_argnames=())
def _sc_gather(idx_flat, codebook):
    B = idx_flat.shape[0]
    V, D = codebook.shape
    b_per_w = B // _NW
    n_chunks = b_per_w // _CHUNK
    C = _CHUNK
    mesh = plsc.VectorSubcoreMesh(core_axis_name="c", subcore_axis_name="s")

    @functools.partial(
        pl.kernel,
        out_type=jax.ShapeDtypeStruct((B, D), jnp.float32),
        mesh=mesh,
        scratch_types=[
            pltpu.VMEM((b_per_w,), jnp.int32),
            pltpu.VMEM((C, D), jnp.float32),
            pltpu.VMEM((C, D), jnp.float32),
            pltpu.SemaphoreType.DMA,
            pltpu.SemaphoreType.DMA,
            pltpu.SemaphoreType.DMA,
            pltpu.SemaphoreType.DMA,
        ],
    )
    def k(table_hbm, idx_hbm, out_hbm, idx_v, rows0, rows1, sg0, sg1, sw0, sw1):
        wid = lax.axis_index("s") * _NUM_CORES + lax.axis_index("c")
        base = wid * b_per_w
        pltpu.sync_copy(idx_hbm.at[pl.ds(base, b_per_w)], idx_v)

        def gather(off, buf, sem):
            pltpu.async_copy(table_hbm.at[idx_v.at[pl.ds(off, C)]], buf, sem)

        def wait_gather(buf, sem):
            pltpu.make_async_copy(table_hbm.at[idx_v.at[pl.ds(0, C)]], buf, sem).wait()

        def write(off, buf, sem):
            pltpu.async_copy(buf, out_hbm.at[pl.ds(base + off, C)], sem)

        def wait_write(buf, sem):
            pltpu.make_async_copy(buf, out_hbm.at[pl.ds(base, C)], sem).wait()

        gather(0, rows0, sg0)
        gather(C, rows1, sg1)

        @pl.loop(0, n_chunks // 2 - 1)
        def _(g):
            off = 2 * g * C
            wait_gather(rows0, sg0)
            write(off, rows0, sw0)
            wait_gather(rows1, sg1)
            write(off + C, rows1, sw1)
            wait_write(rows0, sw0)
            gather(off + 2 * C, rows0, sg0)
            wait_write(rows1, sw1)
            gather(off + 3 * C, rows1, sg1)

        last = (n_chunks - 2) * C
        wait_gather(rows0, sg0)
        write(last, rows0, sw0)
        wait_gather(rows1, sg1)
        write(last + C, rows1, sw1)
        wait_write(rows0, sw0)
        wait_write(rows1, sw1)

    return k(codebook, idx_flat)


def kernel(indices, codebook):
    shape = indices.shape
    idx_flat = indices.reshape(-1).astype(jnp.int32)
    out = _sc_gather(idx_flat, codebook.astype(jnp.float32))
    return out.reshape(*shape, codebook.shape[1])


# speedup vs baseline: 3.1352x; 1.0233x over previous
import functools

import jax
import jax.numpy as jnp
from jax import lax
from jax.experimental import pallas as pl
from jax.experimental.pallas import tpu as pltpu
from jax.experimental.pallas import tpu_sc as plsc

_NUM_CORES = 2
_NUM_SUBCORES = 16
_NW = _NUM_CORES * _NUM_SUBCORES
_CHUNK = 32
_NBUF = 4


@functools.partial(jax.jit, static_argnames=())
def _sc_gather(idx_flat, codebook):
    B = idx_flat.shape[0]
    V, D = codebook.shape
    b_per_w = B // _NW
    n_chunks = b_per_w // _CHUNK
    C = _CHUNK
    mesh = plsc.VectorSubcoreMesh(core_axis_name="c", subcore_axis_name="s")

    nb = _NBUF
    row_bufs = [pltpu.VMEM((C, D), jnp.float32) for _ in range(nb)]
    g_sems = [pltpu.SemaphoreType.DMA for _ in range(nb)]
    w_sems = [pltpu.SemaphoreType.DMA for _ in range(nb)]

    @functools.partial(
        pl.kernel,
        out_type=jax.ShapeDtypeStruct((B, D), jnp.float32),
        mesh=mesh,
        scratch_types=[pltpu.VMEM((b_per_w,), jnp.int32)] + row_bufs + g_sems + w_sems,
    )
    def k(table_hbm, idx_hbm, out_hbm, idx_v, *bufs_and_sems):
        rows = bufs_and_sems[:nb]
        sg = bufs_and_sems[nb : 2 * nb]
        sw = bufs_and_sems[2 * nb : 3 * nb]
        wid = lax.axis_index("s") * _NUM_CORES + lax.axis_index("c")
        base = wid * b_per_w
        pltpu.sync_copy(idx_hbm.at[pl.ds(base, b_per_w)], idx_v)

        def gather(off, b):
            pltpu.async_copy(table_hbm.at[idx_v.at[pl.ds(off, C)]], rows[b], sg[b])

        def wait_gather(b):
            pltpu.make_async_copy(
                table_hbm.at[idx_v.at[pl.ds(0, C)]], rows[b], sg[b]
            ).wait()

        def write(off, b):
            pltpu.async_copy(rows[b], out_hbm.at[pl.ds(base + off, C)], sw[b])

        def wait_write(b):
            pltpu.make_async_copy(rows[b], out_hbm.at[pl.ds(base, C)], sw[b]).wait()

        for b in range(nb):
            gather(b * C, b)

        @pl.loop(0, n_chunks // nb - 1)
        def _(g):
            off = g * nb * C
            for b in range(nb):
                wait_gather(b)
                write(off + b * C, b)
            for b in range(nb):
                wait_write(b)
                gather(off + (nb + b) * C, b)

        last = (n_chunks - nb) * C
        for b in range(nb):
            wait_gather(b)
            write(last + b * C, b)
        for b in range(nb):
            wait_write(b)

    return k(codebook, idx_flat)


def kernel(indices, codebook):
    shape = indices.shape
    idx_flat = indices.reshape(-1).astype(jnp.int32)
    out = _sc_gather(idx_flat, codebook.astype(jnp.float32))
    return out.reshape(*shape, codebook.shape[1])


# speedup vs baseline: 3.1818x; 1.0149x over previous
import functools

import jax
import jax.numpy as jnp
from jax import lax
from jax.experimental import pallas as pl
from jax.experimental.pallas import tpu as pltpu
from jax.experimental.pallas import tpu_sc as plsc

_NUM_CORES = 2
_NUM_SUBCORES = 16
_NW = _NUM_CORES * _NUM_SUBCORES
_CHUNK = 32
_NBUF = 6


@functools.partial(jax.jit, static_argnames=())
def _sc_gather(idx_flat, codebook):
    B = idx_flat.shape[0]
    V, D = codebook.shape
    b_per_w = B // _NW
    n_chunks = b_per_w // _CHUNK
    C = _CHUNK
    mesh = plsc.VectorSubcoreMesh(core_axis_name="c", subcore_axis_name="s")

    nb = _NBUF
    row_bufs = [pltpu.VMEM((C, D), jnp.float32) for _ in range(nb)]
    g_sems = [pltpu.SemaphoreType.DMA for _ in range(nb)]
    w_sems = [pltpu.SemaphoreType.DMA for _ in range(nb)]

    @functools.partial(
        pl.kernel,
        out_type=jax.ShapeDtypeStruct((B, D), jnp.float32),
        mesh=mesh,
        scratch_types=[pltpu.VMEM((b_per_w,), jnp.int32)] + row_bufs + g_sems + w_sems,
    )
    def k(table_hbm, idx_hbm, out_hbm, idx_v, *bufs_and_sems):
        rows = bufs_and_sems[:nb]
        sg = bufs_and_sems[nb : 2 * nb]
        sw = bufs_and_sems[2 * nb : 3 * nb]
        wid = lax.axis_index("s") * _NUM_CORES + lax.axis_index("c")
        base = wid * b_per_w
        pltpu.sync_copy(idx_hbm.at[pl.ds(base, b_per_w)], idx_v)

        def gather(off, b):
            pltpu.async_copy(table_hbm.at[idx_v.at[pl.ds(off, C)]], rows[b], sg[b])

        def wait_gather(b):
            pltpu.make_async_copy(
                table_hbm.at[idx_v.at[pl.ds(0, C)]], rows[b], sg[b]
            ).wait()

        def write(off, b):
            pltpu.async_copy(rows[b], out_hbm.at[pl.ds(base + off, C)], sw[b])

        def wait_write(b):
            pltpu.make_async_copy(rows[b], out_hbm.at[pl.ds(base, C)], sw[b]).wait()

        for b in range(nb):
            gather(b * C, b)

        n_main = (n_chunks // nb) - 1
        tail = n_chunks - (n_main + 1) * nb

        @pl.loop(0, n_main)
        def _(g):
            off = g * nb * C
            for b in range(nb):
                wait_gather(b)
                write(off + b * C, b)
            for b in range(nb):
                wait_write(b)
                gather(off + (nb + b) * C, b)

        last = n_main * nb * C
        for b in range(nb):
            wait_gather(b)
            write(last + b * C, b)
        for b in range(tail):
            wait_write(b)
            gather(last + (nb + b) * C, b)
        for b in range(tail):
            wait_gather(b)
            write(last + (nb + b) * C, b)
        for b in range(nb):
            wait_write(b)

    return k(codebook, idx_flat)


def kernel(indices, codebook):
    shape = indices.shape
    idx_flat = indices.reshape(-1).astype(jnp.int32)
    out = _sc_gather(idx_flat, codebook.astype(jnp.float32))
    return out.reshape(*shape, codebook.shape[1])
